# baseline (speedup 1.0000x reference)
import functools

import jax
import jax.numpy as jnp
from jax import lax
from jax.experimental import pallas as pl
from jax.experimental.pallas import tpu as pltpu
from jax.experimental.pallas import tpu_sc as plsc

N = 10000
E = 320000
D = 128
B = 128
OUT = 40

NC = 2
NS = 16
NW = NC * NS
N_PAD = 10112
RPT = N_PAD // NS
CHUNK = 128
NCHUNK = 2560
E_PAD = NCHUNK * CHUNK
CPT = NCHUNK // NW
CPT0 = 160
HALF = 80
SRC_CHUNKS = NCHUNK
RB = N_PAD // 8

_mesh = plsc.VectorSubcoreMesh(
    core_axis_name="c", subcore_axis_name="s", num_cores=NC, num_subcores=NS)


@functools.partial(
    pl.kernel,
    mesh=_mesh,
    out_type=jax.ShapeDtypeStruct((NC * N_PAD,), jnp.float32),
    scratch_types=[
        pltpu.VMEM_SHARED((N_PAD,), jnp.float32),
        pltpu.VMEM((CHUNK,), jnp.int32),
        pltpu.VMEM((CHUNK,), jnp.float32),
        pltpu.VMEM((RPT,), jnp.float32),
    ],
)
def _sc_deg(dstI, zrow, out, dacc, dst_v, ones_v, bounce_v):
    c = lax.axis_index("c")
    s = lax.axis_index("s")
    pltpu.sync_copy(zrow, bounce_v)
    pltpu.sync_copy(bounce_v, dacc.at[pl.ds(s * RPT, RPT)])
    for i in range(CHUNK // 16):
        ones_v[pl.ds(i * 16, 16)] = jnp.ones((16,), jnp.float32)
    plsc.subcore_barrier()
    base = (c * NS + s) * CPT

    def body(j, carry):
        pltpu.sync_copy(dstI.at[pl.ds((base + j) * CHUNK, CHUNK)], dst_v)
        pltpu.sync_copy(ones_v, dacc.at[dst_v], add=True)
        return carry

    lax.fori_loop(0, CPT, body, 0)
    plsc.subcore_barrier()
    pltpu.sync_copy(dacc.at[pl.ds(s * RPT, RPT)], bounce_v)
    pltpu.sync_copy(bounce_v, out.at[pl.ds(c * N_PAD + s * RPT, RPT)])


_PIECES = ((0, 128), (128, 128), (256, 128), (384, 128), (512, 120))


@functools.partial(
    pl.kernel,
    mesh=_mesh,
    out_type=jax.ShapeDtypeStruct((N_PAD, D), jnp.float32),
    scratch_types=[
        pltpu.VMEM_SHARED((N_PAD, D), jnp.float32),
        pltpu.VMEM((HALF * CHUNK,), jnp.int32),
        pltpu.VMEM((2, CHUNK), jnp.int32),
        pltpu.VMEM((2, CHUNK, D), jnp.float32),
        pltpu.SemaphoreType.DMA((2,)),
    ],
)
def _sc_edge(y, srcI, dstI, ztile, out, acc, srcL, dst_v, rowsB, gsem):
    c = lax.axis_index("c")
    s = lax.axis_index("s")

    @pl.when(c == 0)
    def _run():
        pltpu.sync_copy(ztile, rowsB.at[0])
        for off, ln in _PIECES:
            pltpu.sync_copy(rowsB.at[0, pl.ds(0, ln)],
                            acc.at[pl.ds(s * RPT + off, ln)])
        plsc.subcore_barrier()

        for ph in range(CPT0 // HALF):
            base = s * CPT0 + ph * HALF
            pltpu.sync_copy(
                srcI.at[pl.ds(base * CHUNK, HALF * CHUNK)], srcL)

            def body(j, carry):
                b = j & 1

                @pl.when(j < HALF)
                def _():
                    pltpu.async_copy(
                        y.at[srcL.at[pl.ds(j * CHUNK, CHUNK)]],
                        rowsB.at[b], gsem.at[b])
                    pltpu.sync_copy(
                        dstI.at[pl.ds((base + j) * CHUNK, CHUNK)],
                        dst_v.at[b])

                @pl.when(j > 0)
                def _():
                    pb = 1 - b
                    pltpu.make_async_copy(
                        y.at[srcL.at[pl.ds((j - 1) * CHUNK, CHUNK)]],
                        rowsB.at[pb], gsem.at[pb]).wait()
                    pltpu.sync_copy(rowsB.at[pb], acc.at[dst_v.at[pb]],
                                    add=True)

                return carry

            lax.fori_loop(0, HALF + 1, body, 0)

        plsc.subcore_barrier()
        for off, ln in _PIECES:
            pltpu.sync_copy(acc.at[pl.ds(s * RPT + off, ln)],
                            rowsB.at[0, pl.ds(0, ln)])
            pltpu.sync_copy(rowsB.at[0, pl.ds(0, ln)],
                            out.at[pl.ds(s * RPT + off, ln)])


def _tc_a_body(d0, d1, x, w, dinv_o, y_o):
    di = lax.rsqrt(d0[...] + d1[...] + 1.0)
    dinv_o[...] = di
    y_o[...] = di * jnp.dot(x[...], w[...], preferred_element_type=jnp.float32)


def _tc_b_body(p0, y0, dinv, b0, w, y1_o):
    di = dinv[...]
    h = jnp.maximum(di * (p0[...] + y0[...]) + b0[...], 0.0)
    y1_o[...] = di * jnp.dot(h, w[...], preferred_element_type=jnp.float32)


def _tc_c_body(p0, y1, dinv, b1, batch, wm1, bm1, wm2, bm2, wm3, bm3, out_o):
    h = dinv[...] * (p0[...] + y1[...]) + b1[...]
    gid = lax.broadcasted_iota(jnp.int32, (B, N_PAD), 0)
    oh = (batch[...] == gid).astype(jnp.float32)
    sacc = jnp.dot(oh, h, preferred_element_type=jnp.float32)
    cnt = jnp.sum(oh, axis=1, keepdims=True)
    pooled = sacc / jnp.maximum(cnt, 1.0)
    z = jnp.maximum(jnp.dot(pooled, wm1[...], preferred_element_type=jnp.float32) + bm1[...], 0.0)
    z = jnp.maximum(jnp.dot(z, wm2[...], preferred_element_type=jnp.float32) + bm2[...], 0.0)
    out_o[...] = jnp.dot(z, wm3[...], preferred_element_type=jnp.float32) + bm3[...]


_row = lambda i: (i, 0)
_rep = lambda i: (0, 0)

_tc_a = pl.pallas_call(
    _tc_a_body,
    grid=(N_PAD // RB,),
    in_specs=[
        pl.BlockSpec((RB, 1), _row),
        pl.BlockSpec((RB, 1), _row),
        pl.BlockSpec((RB, D), _row),
        pl.BlockSpec((D, D), _rep),
    ],
    out_specs=[pl.BlockSpec((RB, 1), _row), pl.BlockSpec((RB, D), _row)],
    out_shape=[
        jax.ShapeDtypeStruct((N_PAD, 1), jnp.float32),
        jax.ShapeDtypeStruct((N_PAD, D), jnp.float32),
    ],
)

_tc_b = pl.pallas_call(
    _tc_b_body,
    grid=(N_PAD // RB,),
    in_specs=[
        pl.BlockSpec((RB, D), _row),
        pl.BlockSpec((RB, D), _row),
        pl.BlockSpec((RB, 1), _row),
        pl.BlockSpec((1, D), _rep),
        pl.BlockSpec((D, D), _rep),
    ],
    out_specs=pl.BlockSpec((RB, D), _row),
    out_shape=jax.ShapeDtypeStruct((N_PAD, D), jnp.float32),
)

_tc_c = pl.pallas_call(
    _tc_c_body,
    out_shape=jax.ShapeDtypeStruct((B, OUT), jnp.float32),
)


def kernel(x, edge_index, edge_attr, batch, W0, b0, W1, b1,
           Wm1, bm1, Wm2, bm2, Wm3, bm3):
    src = edge_index[0]
    dst = edge_index[1]
    srcp = jnp.concatenate(
        [src, jnp.zeros((SRC_CHUNKS * CHUNK - E,), jnp.int32)])
    dstp = jnp.concatenate([dst, jnp.full((E_PAD - E,), N, jnp.int32)])
    zrow = jnp.zeros((RPT,), jnp.float32)
    ztile = jnp.zeros((CHUNK, D), jnp.float32)
    x_pad = jnp.pad(x, ((0, N_PAD - N), (0, 0)))
    batchp = jnp.pad(batch, (0, N_PAD - N), constant_values=B).reshape(1, N_PAD)

    degp = _sc_deg(dstp, zrow).reshape(NC, N_PAD)
    d0 = degp[0].reshape(N_PAD, 1)
    d1 = degp[1].reshape(N_PAD, 1)
    dinv, y0 = _tc_a(d0, d1, x_pad, W0)
    p = _sc_edge(y0, srcp, dstp, ztile)
    y1 = _tc_b(p, y0, dinv, b0.reshape(1, D), W1)
    p2 = _sc_edge(y1, srcp, dstp, ztile)
    return _tc_c(p2, y1, dinv, b1.reshape(1, D), batchp,
                 Wm1, bm1.reshape(1, -1), Wm2, bm2.reshape(1, -1),
                 Wm3, bm3.reshape(1, -1))


# speedup vs baseline: 2.3581x; 2.3581x over previous
import functools

import jax
import jax.numpy as jnp
from jax import lax
from jax.experimental import pallas as pl
from jax.experimental.pallas import tpu as pltpu
from jax.experimental.pallas import tpu_sc as plsc

N = 10000
E = 320000
D = 128
B = 128
OUT = 40

NC = 2
NS = 16
NW = NC * NS
N_PAD = 10112
RPT = N_PAD // NS
CHUNK = 96
NCHUNK = 3360
E_PAD = NCHUNK * CHUNK
CPT = NCHUNK // NW
CPT0 = 195
CPT1 = 15
SRC_CHUNKS = NS * CPT0 + (NS - 1) * CPT1 + CPT0
RB = N_PAD // 8

_mesh = plsc.VectorSubcoreMesh(
    core_axis_name="c", subcore_axis_name="s", num_cores=NC, num_subcores=NS)


@functools.partial(
    pl.kernel,
    mesh=_mesh,
    out_type=jax.ShapeDtypeStruct((NC * N_PAD,), jnp.float32),
    scratch_types=[
        pltpu.VMEM_SHARED((N_PAD,), jnp.float32),
        pltpu.VMEM((CHUNK,), jnp.int32),
        pltpu.VMEM((CHUNK,), jnp.float32),
        pltpu.VMEM((RPT,), jnp.float32),
    ],
)
def _sc_deg(dstI, zrow, out, dacc, dst_v, ones_v, bounce_v):
    c = lax.axis_index("c")
    s = lax.axis_index("s")
    pltpu.sync_copy(zrow, bounce_v)
    pltpu.sync_copy(bounce_v, dacc.at[pl.ds(s * RPT, RPT)])
    for i in range(CHUNK // 16):
        ones_v[pl.ds(i * 16, 16)] = jnp.ones((16,), jnp.float32)
    plsc.subcore_barrier()
    base = (c * NS + s) * CPT

    def body(j, carry):
        pltpu.sync_copy(dstI.at[pl.ds((base + j) * CHUNK, CHUNK)], dst_v)
        pltpu.sync_copy(ones_v, dacc.at[dst_v], add=True)
        return carry

    lax.fori_loop(0, CPT, body, 0)
    plsc.subcore_barrier()
    pltpu.sync_copy(dacc.at[pl.ds(s * RPT, RPT)], bounce_v)
    pltpu.sync_copy(bounce_v, out.at[pl.ds(c * N_PAD + s * RPT, RPT)])


_PIECES = ((0, 96), (96, 96), (192, 96), (288, 96), (384, 96), (480, 96),
           (576, 56))


@functools.partial(
    pl.kernel,
    mesh=_mesh,
    out_type=jax.ShapeDtypeStruct((NC, N_PAD, D), jnp.float32),
    scratch_types=[
        pltpu.VMEM_SHARED((N_PAD, D), jnp.float32),
        pltpu.VMEM((CPT0 * CHUNK,), jnp.int32),
        pltpu.VMEM((2, CHUNK), jnp.int32),
        pltpu.VMEM((2, CHUNK, D), jnp.float32),
        pltpu.SemaphoreType.DMA((2,)),
    ],
)
def _sc_edge(y, srcI, dstI, ztile, out, acc, srcL, dst_v, rowsB, gsem):
    c = lax.axis_index("c")
    s = lax.axis_index("s")
    nloc = jnp.where(c == 0, CPT0, CPT1)
    base = jnp.where(c == 0, s * CPT0, NS * CPT0 + s * CPT1)
    pltpu.sync_copy(srcI.at[pl.ds(base * CHUNK, CPT0 * CHUNK)], srcL)
    pltpu.sync_copy(ztile, rowsB.at[0])
    for off, ln in _PIECES:
        pltpu.sync_copy(rowsB.at[0, pl.ds(0, ln)],
                        acc.at[pl.ds(s * RPT + off, ln)])
    plsc.subcore_barrier()

    def _gidx(j):
        return srcL.at[pl.ds(j * CHUNK, CHUNK)]

    def body(j, carry):
        b = j & 1

        @pl.when(j < nloc)
        def _():
            pltpu.async_copy(y.at[_gidx(j)], rowsB.at[b], gsem.at[b])
            pltpu.sync_copy(dstI.at[pl.ds((base + j) * CHUNK, CHUNK)],
                            dst_v.at[b])

        @pl.when(j > 0)
        def _():
            pb = 1 - b
            pltpu.make_async_copy(y.at[_gidx(j - 1)], rowsB.at[pb],
                                  gsem.at[pb]).wait()
            pltpu.sync_copy(rowsB.at[pb], acc.at[dst_v.at[pb]], add=True)

        return carry

    lax.fori_loop(0, nloc + 1, body, 0)
    plsc.subcore_barrier()
    for off, ln in _PIECES:
        pltpu.sync_copy(acc.at[pl.ds(s * RPT + off, ln)],
                        rowsB.at[0, pl.ds(0, ln)])
        pltpu.sync_copy(rowsB.at[0, pl.ds(0, ln)],
                        out.at[c, pl.ds(s * RPT + off, ln)])


def _tc_a_body(d0, d1, x, w, dinv_o, y_o):
    di = lax.rsqrt(d0[...] + d1[...] + 1.0)
    dinv_o[...] = di
    y_o[...] = di * jnp.dot(x[...], w[...], preferred_element_type=jnp.float32)


def _tc_b_body(p0, p1, y0, dinv, b0, w, y1_o):
    di = dinv[...]
    h = jnp.maximum(di * (p0[...] + p1[...] + y0[...]) + b0[...], 0.0)
    y1_o[...] = di * jnp.dot(h, w[...], preferred_element_type=jnp.float32)


def _tc_c_body(p0, p1, y1, dinv, b1, batch, wm1, bm1, wm2, bm2, wm3, bm3, out_o):
    h = dinv[...] * (p0[...] + p1[...] + y1[...]) + b1[...]
    gid = lax.broadcasted_iota(jnp.int32, (B, N_PAD), 0)
    oh = (batch[...] == gid).astype(jnp.float32)
    sacc = jnp.dot(oh, h, preferred_element_type=jnp.float32)
    cnt = jnp.sum(oh, axis=1, keepdims=True)
    pooled = sacc / jnp.maximum(cnt, 1.0)
    z = jnp.maximum(jnp.dot(pooled, wm1[...], preferred_element_type=jnp.float32) + bm1[...], 0.0)
    z = jnp.maximum(jnp.dot(z, wm2[...], preferred_element_type=jnp.float32) + bm2[...], 0.0)
    out_o[...] = jnp.dot(z, wm3[...], preferred_element_type=jnp.float32) + bm3[...]


_row = lambda i: (i, 0)
_rep = lambda i: (0, 0)

_tc_a = pl.pallas_call(
    _tc_a_body,
    grid=(N_PAD // RB,),
    in_specs=[
        pl.BlockSpec((RB, 1), _row),
        pl.BlockSpec((RB, 1), _row),
        pl.BlockSpec((RB, D), _row),
        pl.BlockSpec((D, D), _rep),
    ],
    out_specs=[pl.BlockSpec((RB, 1), _row), pl.BlockSpec((RB, D), _row)],
    out_shape=[
        jax.ShapeDtypeStruct((N_PAD, 1), jnp.float32),
        jax.ShapeDtypeStruct((N_PAD, D), jnp.float32),
    ],
)

_tc_b = pl.pallas_call(
    _tc_b_body,
    grid=(N_PAD // RB,),
    in_specs=[
        pl.BlockSpec((RB, D), _row),
        pl.BlockSpec((RB, D), _row),
        pl.BlockSpec((RB, D), _row),
        pl.BlockSpec((RB, 1), _row),
        pl.BlockSpec((1, D), _rep),
        pl.BlockSpec((D, D), _rep),
    ],
    out_specs=pl.BlockSpec((RB, D), _row),
    out_shape=jax.ShapeDtypeStruct((N_PAD, D), jnp.float32),
)

_tc_c = pl.pallas_call(
    _tc_c_body,
    out_shape=jax.ShapeDtypeStruct((B, OUT), jnp.float32),
)


def kernel(x, edge_index, edge_attr, batch, W0, b0, W1, b1,
           Wm1, bm1, Wm2, bm2, Wm3, bm3):
    src = edge_index[0]
    dst = edge_index[1]
    srcp = jnp.concatenate(
        [src, jnp.zeros((SRC_CHUNKS * CHUNK - E,), jnp.int32)])
    dstp = jnp.concatenate([dst, jnp.full((E_PAD - E,), N, jnp.int32)])
    zrow = jnp.zeros((RPT,), jnp.float32)
    ztile = jnp.zeros((CHUNK, D), jnp.float32)
    x_pad = jnp.pad(x, ((0, N_PAD - N), (0, 0)))
    batchp = jnp.pad(batch, (0, N_PAD - N), constant_values=B).reshape(1, N_PAD)

    degp = _sc_deg(dstp, zrow).reshape(NC, N_PAD)
    d0 = degp[0].reshape(N_PAD, 1)
    d1 = degp[1].reshape(N_PAD, 1)
    dinv, y0 = _tc_a(d0, d1, x_pad, W0)
    p = _sc_edge(y0, srcp, dstp, ztile)
    y1 = _tc_b(p[0], p[1], y0, dinv, b0.reshape(1, D), W1)
    p2 = _sc_edge(y1, srcp, dstp, ztile)
    return _tc_c(p2[0], p2[1], y1, dinv, b1.reshape(1, D), batchp,
                 Wm1, bm1.reshape(1, -1), Wm2, bm2.reshape(1, -1),
                 Wm3, bm3.reshape(1, -1))


# speedup vs baseline: 2.4076x; 1.0210x over previous
import functools

import jax
import jax.numpy as jnp
from jax import lax
from jax.experimental import pallas as pl
from jax.experimental.pallas import tpu as pltpu
from jax.experimental.pallas import tpu_sc as plsc

N = 10000
E = 320000
D = 128
B = 128
OUT = 40

NC = 2
NS = 16
NW = NC * NS
N_PAD = 10112
RPT = N_PAD // NS
CHUNK = 96
NCHUNK = 3360
E_PAD = NCHUNK * CHUNK
CPT = NCHUNK // NW
CPT0 = 195
CPT1 = 15
SRC_CHUNKS = NS * CPT0 + (NS - 1) * CPT1 + CPT0
RB = N_PAD // 8

_mesh = plsc.VectorSubcoreMesh(
    core_axis_name="c", subcore_axis_name="s", num_cores=NC, num_subcores=NS)


@functools.partial(
    pl.kernel,
    mesh=_mesh,
    out_type=jax.ShapeDtypeStruct((NC * N_PAD,), jnp.float32),
    scratch_types=[
        pltpu.VMEM_SHARED((N_PAD,), jnp.float32),
        pltpu.VMEM((2, CHUNK), jnp.int32),
        pltpu.VMEM((CHUNK,), jnp.float32),
        pltpu.VMEM((RPT,), jnp.float32),
        pltpu.SemaphoreType.DMA((2,)),
    ],
)
def _sc_deg(dstI, zrow, out, dacc, dst_v, ones_v, bounce_v, isem):
    c = lax.axis_index("c")
    s = lax.axis_index("s")
    pltpu.sync_copy(zrow, bounce_v)
    pltpu.sync_copy(bounce_v, dacc.at[pl.ds(s * RPT, RPT)])
    for i in range(CHUNK // 16):
        ones_v[pl.ds(i * 16, 16)] = jnp.ones((16,), jnp.float32)
    plsc.subcore_barrier()
    base = (c * NS + s) * CPT

    def body(j, carry):
        b = j & 1

        @pl.when(j < CPT)
        def _():
            pltpu.async_copy(dstI.at[pl.ds((base + j) * CHUNK, CHUNK)],
                             dst_v.at[b], isem.at[b])

        @pl.when(j > 0)
        def _():
            pb = 1 - b
            pltpu.make_async_copy(
                dstI.at[pl.ds((base + j - 1) * CHUNK, CHUNK)],
                dst_v.at[pb], isem.at[pb]).wait()
            pltpu.sync_copy(ones_v, dacc.at[dst_v.at[pb]], add=True)

        return carry

    lax.fori_loop(0, CPT + 1, body, 0)
    plsc.subcore_barrier()
    pltpu.sync_copy(dacc.at[pl.ds(s * RPT, RPT)], bounce_v)
    pltpu.sync_copy(bounce_v, out.at[pl.ds(c * N_PAD + s * RPT, RPT)])


_PIECES = ((0, 96), (96, 96), (192, 96), (288, 96), (384, 96), (480, 96),
           (576, 56))


@functools.partial(
    pl.kernel,
    mesh=_mesh,
    out_type=jax.ShapeDtypeStruct((NC, N_PAD, D), jnp.float32),
    scratch_types=[
        pltpu.VMEM_SHARED((N_PAD, D), jnp.float32),
        pltpu.VMEM((CPT0 * CHUNK,), jnp.int32),
        pltpu.VMEM((2, CHUNK), jnp.int32),
        pltpu.VMEM((2, CHUNK, D), jnp.float32),
        pltpu.SemaphoreType.DMA((2,)),
    ],
)
def _sc_edge(y, srcI, dstI, ztile, out, acc, srcL, dst_v, rowsB, gsem):
    c = lax.axis_index("c")
    s = lax.axis_index("s")
    nloc = jnp.where(c == 0, CPT0, CPT1)
    base = jnp.where(c == 0, s * CPT0, NS * CPT0 + s * CPT1)
    pltpu.sync_copy(srcI.at[pl.ds(base * CHUNK, CPT0 * CHUNK)], srcL)
    pltpu.sync_copy(ztile, rowsB.at[0])
    for off, ln in _PIECES:
        pltpu.sync_copy(rowsB.at[0, pl.ds(0, ln)],
                        acc.at[pl.ds(s * RPT + off, ln)])
    plsc.subcore_barrier()

    def _gidx(j):
        return srcL.at[pl.ds(j * CHUNK, CHUNK)]

    def body(j, carry):
        b = j & 1

        @pl.when(j < nloc)
        def _():
            pltpu.async_copy(y.at[_gidx(j)], rowsB.at[b], gsem.at[b])
            pltpu.sync_copy(dstI.at[pl.ds((base + j) * CHUNK, CHUNK)],
                            dst_v.at[b])

        @pl.when(j > 0)
        def _():
            pb = 1 - b
            pltpu.make_async_copy(y.at[_gidx(j - 1)], rowsB.at[pb],
                                  gsem.at[pb]).wait()
            pltpu.sync_copy(rowsB.at[pb], acc.at[dst_v.at[pb]], add=True)

        return carry

    lax.fori_loop(0, nloc + 1, body, 0)
    plsc.subcore_barrier()
    for off, ln in _PIECES:
        pltpu.sync_copy(acc.at[pl.ds(s * RPT + off, ln)],
                        rowsB.at[0, pl.ds(0, ln)])
        pltpu.sync_copy(rowsB.at[0, pl.ds(0, ln)],
                        out.at[c, pl.ds(s * RPT + off, ln)])


def _tc_mm_body(x, w, xw_o):
    xw_o[...] = jnp.dot(x[...], w[...], preferred_element_type=jnp.float32)


def _tc_scale_body(d0, d1, xw, dinv_o, y_o):
    di = lax.rsqrt(d0[...] + d1[...] + 1.0)
    dinv_o[...] = di
    y_o[...] = di * xw[...]


def _tc_b_body(p0, p1, y0, dinv, b0, w, y1_o):
    di = dinv[...]
    h = jnp.maximum(di * (p0[...] + p1[...] + y0[...]) + b0[...], 0.0)
    y1_o[...] = di * jnp.dot(h, w[...], preferred_element_type=jnp.float32)


def _tc_c_body(p0, p1, y1, dinv, b1, batch, wm1, bm1, wm2, bm2, wm3, bm3, out_o):
    h = dinv[...] * (p0[...] + p1[...] + y1[...]) + b1[...]
    gid = lax.broadcasted_iota(jnp.int32, (B, N_PAD), 0)
    oh = (batch[...] == gid).astype(jnp.float32)
    sacc = jnp.dot(oh, h, preferred_element_type=jnp.float32)
    cnt = jnp.sum(oh, axis=1, keepdims=True)
    pooled = sacc / jnp.maximum(cnt, 1.0)
    z = jnp.maximum(jnp.dot(pooled, wm1[...], preferred_element_type=jnp.float32) + bm1[...], 0.0)
    z = jnp.maximum(jnp.dot(z, wm2[...], preferred_element_type=jnp.float32) + bm2[...], 0.0)
    out_o[...] = jnp.dot(z, wm3[...], preferred_element_type=jnp.float32) + bm3[...]


_row = lambda i: (i, 0)
_rep = lambda i: (0, 0)

_tc_mm = pl.pallas_call(
    _tc_mm_body,
    grid=(N_PAD // RB,),
    in_specs=[
        pl.BlockSpec((RB, D), _row),
        pl.BlockSpec((D, D), _rep),
    ],
    out_specs=pl.BlockSpec((RB, D), _row),
    out_shape=jax.ShapeDtypeStruct((N_PAD, D), jnp.float32),
)

_tc_scale = pl.pallas_call(
    _tc_scale_body,
    grid=(N_PAD // RB,),
    in_specs=[
        pl.BlockSpec((RB, 1), _row),
        pl.BlockSpec((RB, 1), _row),
        pl.BlockSpec((RB, D), _row),
    ],
    out_specs=[pl.BlockSpec((RB, 1), _row), pl.BlockSpec((RB, D), _row)],
    out_shape=[
        jax.ShapeDtypeStruct((N_PAD, 1), jnp.float32),
        jax.ShapeDtypeStruct((N_PAD, D), jnp.float32),
    ],
)

_tc_b = pl.pallas_call(
    _tc_b_body,
    grid=(N_PAD // RB,),
    in_specs=[
        pl.BlockSpec((RB, D), _row),
        pl.BlockSpec((RB, D), _row),
        pl.BlockSpec((RB, D), _row),
        pl.BlockSpec((RB, 1), _row),
        pl.BlockSpec((1, D), _rep),
        pl.BlockSpec((D, D), _rep),
    ],
    out_specs=pl.BlockSpec((RB, D), _row),
    out_shape=jax.ShapeDtypeStruct((N_PAD, D), jnp.float32),
)

_tc_c = pl.pallas_call(
    _tc_c_body,
    out_shape=jax.ShapeDtypeStruct((B, OUT), jnp.float32),
)


def kernel(x, edge_index, edge_attr, batch, W0, b0, W1, b1,
           Wm1, bm1, Wm2, bm2, Wm3, bm3):
    src = edge_index[0]
    dst = edge_index[1]
    srcp = jnp.concatenate(
        [src, jnp.zeros((SRC_CHUNKS * CHUNK - E,), jnp.int32)])
    dstp = jnp.concatenate([dst, jnp.full((E_PAD - E,), N, jnp.int32)])
    zrow = jnp.zeros((RPT,), jnp.float32)
    ztile = jnp.zeros((CHUNK, D), jnp.float32)
    x_pad = jnp.pad(x, ((0, N_PAD - N), (0, 0)))
    batchp = jnp.pad(batch, (0, N_PAD - N), constant_values=B).reshape(1, N_PAD)

    degp = _sc_deg(dstp, zrow).reshape(NC, N_PAD)
    xw0 = _tc_mm(x_pad, W0)
    d0 = degp[0].reshape(N_PAD, 1)
    d1 = degp[1].reshape(N_PAD, 1)
    dinv, y0 = _tc_scale(d0, d1, xw0)
    p = _sc_edge(y0, srcp, dstp, ztile)
    y1 = _tc_b(p[0], p[1], y0, dinv, b0.reshape(1, D), W1)
    p2 = _sc_edge(y1, srcp, dstp, ztile)
    return _tc_c(p2[0], p2[1], y1, dinv, b1.reshape(1, D), batchp,
                 Wm1, bm1.reshape(1, -1), Wm2, bm2.reshape(1, -1),
                 Wm3, bm3.reshape(1, -1))
